# Initial kernel scaffold; baseline (speedup 1.0000x reference)
import jax
import jax.numpy as jnp
from jax.experimental import pallas as pl


def kernel(input_image_re, input_image_im, input_mask, output_re, output_im, w1_re, w2_re, w1_im, w2_im):
    raise NotImplementedError("write your pallas kernel here")


# speedup vs baseline: 63.5106x; 63.5106x over previous
import numpy as np
import jax
import jax.numpy as jnp
from jax import lax
from jax.experimental import pallas as pl
from jax.experimental.pallas import tpu as pltpu

_W = 512
_F = 16
_NR = 96
_ROWS = 8
_NBLK = _W // _ROWS
_N = _ROWS * _W

_HIGH = lax.Precision.HIGHEST


def _inv_counts():
    xi = np.arange(-_W // 2, _W // 2, dtype=np.float64)
    X, Y = np.meshgrid(xi, xi, indexing='ij')
    rid = (np.sqrt(X * X + Y * Y) // 4).astype(np.int64)
    cnt = np.bincount(rid.ravel(), minlength=_NR).astype(np.float64)
    inv = np.where(cnt > 0, 1.0 / np.maximum(cnt, 1.0), 0.0)
    return inv.astype(np.float32).reshape(_NR, 1)

_INV_NP = _inv_counts()

_E_NP = np.zeros((_F, 2 * _F), np.float32)
_O_NP = np.zeros((_F, 2 * _F), np.float32)
for _k in range(_F):
    _E_NP[_k, 2 * _k] = 1.0
    _O_NP[_k, 2 * _k + 1] = 1.0


def _ring_onehot_bool(row0):
    shp = (_ROWS, _W, _NR)
    ii = lax.broadcasted_iota(jnp.int32, shp, 0) + row0 - _W // 2
    jj = lax.broadcasted_iota(jnp.int32, shp, 1) - _W // 2
    kk = lax.broadcasted_iota(jnp.int32, shp, 2)
    r2 = ii * ii + jj * jj
    k2 = kk * kk
    lo = k2 * 16
    hi = (k2 + 2 * kk + 1) * 16
    return (r2 >= lo) & (r2 < hi)


def _pass1(ire_ref, iim_ref, m_ref, inv_ref, w1r_ref, w1i_ref,
           e_ref, o_ref, s_ref, acc_ref):
    pid = pl.program_id(0)

    @pl.when(pid == 0)
    def _():
        acc_ref[...] = jnp.zeros_like(acc_ref)

    unc3 = (m_ref[...] == 0).astype(jnp.float32)
    oh = _ring_onehot_bool(pid * _ROWS).astype(jnp.float32)
    ohm = (oh * unc3).reshape(_N, _NR)
    a_re = ire_ref[...].reshape(_N, _F)
    a_im = iim_ref[...].reshape(_N, _F)
    u = jnp.concatenate([a_re, a_im], axis=1)
    acc_ref[...] += lax.dot_general(
        ohm, u, (((0,), (0,)), ((), ())),
        preferred_element_type=jnp.float32)

    @pl.when(pid == _NBLK - 1)
    def _():
        inv = inv_ref[...]
        sum_re = acc_ref[:, :_F]
        sum_im = acc_ref[:, _F:]
        w1r = w1r_ref[...]
        w1i = w1i_ref[...]
        s_re = (jnp.dot(sum_re, w1r, precision=_HIGH)
                - jnp.dot(sum_im, w1i, precision=_HIGH)) * inv
        s_im = (jnp.dot(sum_re, w1i, precision=_HIGH)
                + jnp.dot(sum_im, w1r, precision=_HIGH)) * inv
        s_ref[...] = (jnp.dot(s_re, e_ref[...], precision=_HIGH)
                      + jnp.dot(s_im, o_ref[...], precision=_HIGH))


def _pass2(ire_ref, iim_ref, m_ref, s_ref,
           w1r_ref, w1i_ref, w2r_ref, w2i_ref, e_ref, o_ref, o_out):
    pid = pl.program_id(0)
    e = e_ref[...]
    o = o_ref[...]
    w1r = w1r_ref[...]; w1i = w1i_ref[...]
    w2r = w2r_ref[...]; w2i = w2i_ref[...]
    wa = (jnp.dot(w1r, e, precision=_HIGH)
          + jnp.dot(w1i, o, precision=_HIGH))
    wb = (jnp.dot(w1r, o, precision=_HIGH)
          - jnp.dot(w1i, e, precision=_HIGH))
    wc = (jnp.dot(w2r, e, precision=_HIGH)
          + jnp.dot(w2i, o, precision=_HIGH))
    wd = (jnp.dot(w2r, o, precision=_HIGH)
          - jnp.dot(w2i, e, precision=_HIGH))

    a_re = ire_ref[...].reshape(_N, _F)
    a_im = iim_ref[...].reshape(_N, _F)
    x1 = (jnp.dot(a_re, wa, precision=_HIGH)
          + jnp.dot(a_im, wb, precision=_HIGH))
    x2 = (jnp.dot(a_re, wc, precision=_HIGH)
          + jnp.dot(a_im, wd, precision=_HIGH))

    oh = _ring_onehot_bool(pid * _ROWS).astype(jnp.float32)
    sg = jnp.dot(oh.reshape(_N, _NR), s_ref[...],
                 precision=_HIGH)

    x1_3 = x1.reshape(_ROWS, _W, 2 * _F)
    x2_3 = x2.reshape(_ROWS, _W, 2 * _F)
    sg_3 = sg.reshape(_ROWS, _W, 2 * _F)
    unc3 = (m_ref[...] == 0).astype(jnp.float32)
    corm3 = (m_ref[...] == 1).astype(jnp.float32)
    o_out[...] = unc3 * (0.5 * (x1_3 + sg_3)) + corm3 * (x2_3 - sg_3)


def _full(shape):
    nd = len(shape)
    return pl.BlockSpec(shape, lambda i: (0,) * nd)


def kernel(input_image_re, input_image_im, input_mask, output_re, output_im,
           w1_re, w2_re, w1_im, w2_im):
    del output_re, output_im
    inv = jnp.asarray(_INV_NP)
    e_m = jnp.asarray(_E_NP)
    o_m = jnp.asarray(_O_NP)
    mask3 = input_mask.reshape(_W, _W, 1)

    img_spec = pl.BlockSpec((_ROWS, _W, _F), lambda i: (i, 0, 0))
    msk_spec = pl.BlockSpec((_ROWS, _W, 1), lambda i: (i, 0, 0))

    s_table = pl.pallas_call(
        _pass1,
        grid=(_NBLK,),
        in_specs=[img_spec, img_spec, msk_spec,
                  _full((_NR, 1)), _full((_F, _F)), _full((_F, _F)),
                  _full((_F, 2 * _F)), _full((_F, 2 * _F))],
        out_specs=_full((_NR, 2 * _F)),
        out_shape=jax.ShapeDtypeStruct((_NR, 2 * _F), jnp.float32),
        scratch_shapes=[pltpu.VMEM((_NR, 2 * _F), jnp.float32)],
    )(input_image_re, input_image_im, mask3, inv, w1_re, w1_im, e_m, o_m)

    out32 = pl.pallas_call(
        _pass2,
        grid=(_NBLK,),
        in_specs=[img_spec, img_spec, msk_spec, _full((_NR, 2 * _F)),
                  _full((_F, _F)), _full((_F, _F)),
                  _full((_F, _F)), _full((_F, _F)),
                  _full((_F, 2 * _F)), _full((_F, 2 * _F))],
        out_specs=pl.BlockSpec((_ROWS, _W, 2 * _F), lambda i: (i, 0, 0)),
        out_shape=jax.ShapeDtypeStruct((_W, _W, 2 * _F), jnp.float32),
    )(input_image_re, input_image_im, mask3, s_table,
      w1_re, w1_im, w2_re, w2_im, e_m, o_m)

    return out32.reshape(_W, _W, _F, 2)
